# baseline (speedup 1.0000x reference)
import functools

import jax
import jax.numpy as jnp
from jax.experimental import pallas as pl
from jax.experimental.pallas import tpu as pltpu
from jax.experimental.pallas import tpu_sc as plsc

_MESH = plsc.ScalarSubcoreMesh(axis_name="c", num_cores=1)


@functools.partial(
    pl.kernel,
    mesh=_MESH,
    out_type=jax.ShapeDtypeStruct((1,), jnp.float32),
    scratch_types=[
        pltpu.SMEM((96,), jnp.float32),
        pltpu.SMEM((1,), jnp.float32),
    ],
    compiler_params=pltpu.CompilerParams(needs_layout_passes=False),
)
def _akima_scs(p_hbm, out_hbm, p_s, o_s):
    pltpu.sync_copy(p_hbm, p_s)
    x = p_s[0]
    cnt = jnp.int32(0)
    for j in range(16):
        cnt = cnt + jnp.where(p_s[16 + j] <= x, jnp.int32(1), jnp.int32(0))
    i = jnp.clip(cnt - 1, 0, 14)
    bx = x - p_s[16 + i]
    c0 = p_s[32 + i]
    c1 = p_s[48 + i]
    c2 = p_s[64 + i]
    c3 = p_s[80 + i]
    v = c3 + bx * (c2 + bx * (c1 + bx * c0))
    o_s[0] = jnp.where(cnt < 16, v, jnp.float32(0.0))
    pltpu.sync_copy(o_s, out_hbm)


def kernel(b, xs, c):
    packed = jnp.concatenate(
        [
            jnp.broadcast_to(b, (1, 16)),
            xs[None, :],
            jnp.pad(c, ((0, 0), (0, 1))),
        ],
        axis=0,
    ).reshape(-1)
    return _akima_scs(packed)[0]
